# baseline (speedup 1.0000x reference)
import functools

import jax
import jax.numpy as jnp
from jax import lax
from jax.experimental import pallas as pl
from jax.experimental.pallas import tpu as pltpu
from jax.experimental.pallas import tpu_sc as plsc

N = 2048
R = 3
P = 2
E = 32768
D_IN = 128
D_H = 64
EPS = 1e-6

BM = 256
NBLK = N // BM


def _stage1_body(a_ref, wa_ref, wb_ref, wc_ref, ra_ref, rb_ref, rb2_ref):
    a = [a_ref[r] for r in range(R)]
    u = (a[0] + a[1] + a[2]) > 0.0
    for p in range(P):
        sa = wa_ref[p, 0] * a[0] + wa_ref[p, 1] * a[1] + wa_ref[p, 2] * a[2]
        ea = jnp.where(u, jnp.exp(sa), 0.0)
        ra_ref[p] = (ea / (jnp.sum(ea, axis=-1, keepdims=True) + EPS)).astype(jnp.bfloat16)
        sb = wb_ref[p, 0] * a[0] + wb_ref[p, 1] * a[1] + wb_ref[p, 2] * a[2]
        eb = jnp.where(u, jnp.exp(sb), 0.0)
        rb_ref[p] = (eb / (jnp.sum(eb, axis=-1, keepdims=True) + EPS)).astype(jnp.bfloat16)
        rb2_ref[p] = (wc_ref[p, 0] * a[0] + wc_ref[p, 1] * a[1]
                      + wc_ref[p, 2] * a[2]).astype(jnp.bfloat16)


def _stage1(A, W_a, W_b, W_c, interpret=False):
    big = jax.ShapeDtypeStruct((P, N, N), jnp.bfloat16)
    wspec = pl.BlockSpec(memory_space=pltpu.SMEM)
    return pl.pallas_call(
        _stage1_body,
        grid=(NBLK,),
        in_specs=[
            pl.BlockSpec((R, BM, N), lambda i: (0, i, 0)),
            wspec, wspec, wspec,
        ],
        out_specs=[
            pl.BlockSpec((P, BM, N), lambda i: (0, i, 0)),
            pl.BlockSpec((P, BM, N), lambda i: (0, i, 0)),
            pl.BlockSpec((P, BM, N), lambda i: (0, i, 0)),
        ],
        out_shape=[big, big, big],
        compiler_params=pltpu.CompilerParams(
            dimension_semantics=("parallel",)),
        interpret=interpret,
    )(A, W_a, W_b, W_c)


def _matmul_body(lhs_ref, rhs_ref, out_ref):
    acc = lax.dot_general(lhs_ref[0], rhs_ref[0], (((1,), (0,)), ((), ())),
                          preferred_element_type=jnp.float32)
    out_ref[0] = acc.astype(jnp.bfloat16)


def _pmatmul(lhs, rhs, interpret=False):
    return pl.pallas_call(
        _matmul_body,
        grid=(P, NBLK),
        in_specs=[
            pl.BlockSpec((1, BM, N), lambda p, i: (p, i, 0)),
            pl.BlockSpec((1, N, N), lambda p, i: (p, 0, 0)),
        ],
        out_specs=pl.BlockSpec((1, BM, N), lambda p, i: (p, i, 0)),
        out_shape=jax.ShapeDtypeStruct((P, N, N), jnp.bfloat16),
        compiler_params=pltpu.CompilerParams(
            dimension_semantics=("parallel", "arbitrary")),
        interpret=interpret,
    )(lhs, rhs)


def _hconv_body(ra_ref, n1_ref, n2_ref, feat_ref, wg_ref, h_ref):
    i = pl.program_id(1)
    x = jnp.dot(feat_ref[...], wg_ref[0],
                preferred_element_type=jnp.float32,
                precision=lax.Precision.HIGHEST)
    xb = x.astype(jnp.bfloat16)
    col = lax.broadcasted_iota(jnp.int32, (BM, N), 1)
    rowg = lax.broadcasted_iota(jnp.int32, (BM, N), 0) + i * BM
    offdiag = col != rowg

    @pl.when(i == 0)
    def _():
        h_ref[...] = jnp.zeros_like(h_ref)

    for j, ref in enumerate((ra_ref, n1_ref, n2_ref)):
        c = ref[0].astype(jnp.float32)
        c = jnp.where(offdiag, c, 0.0)
        cn = c / (jnp.sum(c, axis=-1, keepdims=True) + EPS)
        contrib = lax.dot_general(cn.astype(jnp.bfloat16), xb,
                                  (((0,), (0,)), ((), ())),
                                  preferred_element_type=jnp.float32)
        h_ref[0, j] += contrib


def _hconv(RA, new1, new2, feat, Wg, interpret=False):
    return pl.pallas_call(
        _hconv_body,
        grid=(P, NBLK),
        in_specs=[
            pl.BlockSpec((1, BM, N), lambda p, i: (p, i, 0)),
            pl.BlockSpec((1, BM, N), lambda p, i: (p, i, 0)),
            pl.BlockSpec((1, BM, N), lambda p, i: (p, i, 0)),
            pl.BlockSpec((BM, D_IN), lambda p, i: (i, 0)),
            pl.BlockSpec((1, D_IN, D_H), lambda p, i: (p, 0, 0)),
        ],
        out_specs=pl.BlockSpec((1, 3, N, D_H), lambda p, i: (p, 0, 0, 0)),
        out_shape=jax.ShapeDtypeStruct((P, 3, N, D_H), jnp.float32),
        compiler_params=pltpu.CompilerParams(
            dimension_semantics=("parallel", "arbitrary")),
        interpret=interpret,
    )(RA, new1, new2, feat, Wg)


def _attn_body(h_ref, bg_ref, law1_ref, lab1_ref, law2_ref,
               maw1_ref, mab1_ref, maw2_ref, out_ref):
    mfs = []
    for p in range(P):
        hs = [jax.nn.relu(h_ref[p, j] + bg_ref[p][None, :]) for j in range(3)]
        cols = []
        for h in hs:
            s = jnp.sum(h * law1_ref[p][None, :], axis=-1, keepdims=True)
            cols.append(jax.nn.relu(jnp.tanh(s + lab1_ref[p]) * law2_ref[p]))
        a = jnp.concatenate(cols, axis=1)
        a = a - jnp.max(a, axis=1, keepdims=True)
        ea = jnp.exp(a)
        b = ea / jnp.sum(ea, axis=1, keepdims=True)
        mf = sum(hs[j] * b[:, j:j + 1] for j in range(3))
        mfs.append(mf)
    mcols = []
    for mf in mfs:
        s = jnp.sum(mf * maw1_ref[0][None, :], axis=-1, keepdims=True)
        mcols.append(jax.nn.relu(jnp.tanh(s + mab1_ref[0]) * maw2_ref[0]))
    ma = jnp.concatenate(mcols, axis=1)
    ma = ma - jnp.max(ma, axis=1, keepdims=True)
    em = jnp.exp(ma)
    mb = em / jnp.sum(em, axis=1, keepdims=True)
    out_ref[...] = sum(mfs[p] * mb[:, p:p + 1] for p in range(P))


BA = 512


def _attn(hacc, bg, la_w1, la_b1, la_w2, ma_w1, ma_b1, ma_w2, interpret=False):
    sspec = pl.BlockSpec(memory_space=pltpu.SMEM)
    return pl.pallas_call(
        _attn_body,
        grid=(N // BA,),
        in_specs=[
            pl.BlockSpec((P, 3, BA, D_H), lambda i: (0, 0, i, 0)),
            pl.BlockSpec((P, D_H), lambda i: (0, 0)),
            pl.BlockSpec((P, D_H), lambda i: (0, 0)),
            sspec,
            sspec,
            pl.BlockSpec((1, D_H), lambda i: (0, 0)),
            sspec,
            sspec,
        ],
        out_specs=pl.BlockSpec((BA, D_H), lambda i: (i, 0)),
        out_shape=jax.ShapeDtypeStruct((N, D_H), jnp.float32),
        compiler_params=pltpu.CompilerParams(
            dimension_semantics=("parallel",)),
        interpret=interpret,
    )(hacc, bg, la_w1, la_b1, la_w2, ma_w1, ma_b1, ma_w2)


WS = 1 << 20
NWIN_A = R * N * N // WS
NTILE = 16
EPT = E // NTILE
NROW = EPT // 128
PT = WS // NTILE
ZCH = 8192
DELTA = 2.0 ** -20


NBUF = 16


def _sc_scatter_body(ei_hbm, ew_hbm, a_hbm,
                     flat_v, w_v, d_v, s_v, iv_bufs, zero_v, win, sem):
    cid = lax.axis_index("c")
    tid = lax.axis_index("s")
    base_e = tid * EPT

    for r in range(R):
        pltpu.sync_copy(ei_hbm.at[pl.ds(r * 2 * E + E + base_e, EPT)], d_v)
        pltpu.sync_copy(ei_hbm.at[pl.ds(r * 2 * E + base_e, EPT)], s_v)
        pltpu.sync_copy(ew_hbm.at[pl.ds(r * E + base_e, EPT)],
                        w_v.at[pl.ds(r * EPT, EPT)])

        @pl.loop(0, EPT // 16)
        def _(c, r=r):
            sl = pl.ds(c * 16, 16)
            fsl = pl.ds(r * EPT + c * 16, 16)
            flat_v[fsl] = d_v[sl] * N + s_v[sl]
            w_v[fsl] = w_v[fsl] + DELTA

    @pl.loop(0, ZCH // 16)
    def _(c):
        zero_v[pl.ds(c * 16, 16)] = jnp.zeros((16,), jnp.float32)

    def zero_window():
        @pl.loop(0, PT // ZCH)
        def _(z):
            pltpu.sync_copy(zero_v, win.at[pl.ds(tid * PT + z * ZCH, ZCH)])

    def stream_rel(rbase, w0):
        @pl.loop(0, NROW, step=NBUF)
        def _(i):
            handles = []
            for b in range(NBUF):
                ib, vb = iv_bufs[2 * b], iv_bufs[2 * b + 1]
                for k in range(8):
                    sl = pl.ds(k * 16, 16)
                    fsl = pl.ds(rbase + (i + b) * 128 + k * 16, 16)
                    adj = flat_v[fsl] - w0
                    ok = (adj >= 0) & (adj < WS)
                    ib[sl] = jnp.where(ok, adj, WS)
                    vb[sl] = w_v[fsl]
                handles.append(pltpu.async_copy(
                    vb, win.at[ib], sem, add=True))
            for h in handles:
                h.wait()

    @pl.loop(0, NWIN_A // 2)
    def _(iw):
        w = cid * (NWIN_A // 2) + iw
        r = w // 4
        q = w - r * 4
        zero_window()
        plsc.subcore_barrier()

        @pl.loop(0, NTILE)
        def _(t):
            @pl.when(tid == t)
            def _():
                stream_rel(r * EPT, q * WS)
            plsc.subcore_barrier()
        pltpu.sync_copy(win.at[pl.ds(tid * PT, PT)],
                        a_hbm.at[pl.ds(w * WS + tid * PT, PT)])


def _scatter_dense(edge_index, edge_weight):
    mesh = plsc.VectorSubcoreMesh(core_axis_name="c", subcore_axis_name="s")
    kern = pl.kernel(
        _sc_scatter_body,
        out_type=jax.ShapeDtypeStruct((R * N * N,), jnp.float32),
        mesh=mesh,
        scratch_types=[
            pltpu.VMEM((R * EPT,), jnp.int32),
            pltpu.VMEM((R * EPT,), jnp.float32),
            pltpu.VMEM((EPT,), jnp.int32),
            pltpu.VMEM((EPT,), jnp.int32),
            [pltpu.VMEM((128,), jnp.int32) if b % 2 == 0
             else pltpu.VMEM((128,), jnp.float32) for b in range(2 * NBUF)],
            pltpu.VMEM((ZCH,), jnp.float32),
            pltpu.VMEM_SHARED((WS + 16,), jnp.float32),
            pltpu.SemaphoreType.DMA,
        ])
    return kern(edge_index.reshape(R * 2 * E), edge_weight.reshape(R * E))


def kernel(feat, edge_index, edge_weight, W_a, W_b, W_c, Wg, bg,
           la_w1, la_b1, la_w2, ma_w1, ma_b1, ma_w2, _interpret=False):
    A_flat = _scatter_dense(edge_index, edge_weight)
    A = A_flat.reshape(R, N, N)
    RA, RB, RB2 = _stage1(A, W_a, W_b, W_c, interpret=_interpret)
    new1 = _pmatmul(RA, RB, interpret=_interpret)
    new2 = _pmatmul(new1, RB2, interpret=_interpret)
    hacc = _hconv(RA, new1, new2, feat, Wg, interpret=_interpret)
    out = _attn(hacc, bg, la_w1, la_b1, la_w2,
                ma_w1.reshape(1, D_H), ma_b1.reshape(1), ma_w2.reshape(1),
                interpret=_interpret)
    return out


# speedup vs baseline: 1.0348x; 1.0348x over previous
import functools

import jax
import jax.numpy as jnp
from jax import lax
from jax.experimental import pallas as pl
from jax.experimental.pallas import tpu as pltpu
from jax.experimental.pallas import tpu_sc as plsc

N = 2048
R = 3
P = 2
E = 32768
D_IN = 128
D_H = 64
EPS = 1e-6

BM = 256
NBLK = N // BM


def _stage1_body(a_ref, wa_ref, wb_ref, wc_ref, ra_ref, rb_ref, rb2_ref):
    a = [a_ref[r] for r in range(R)]
    u = (a[0] + a[1] + a[2]) > 0.0
    for p in range(P):
        sa = wa_ref[p, 0] * a[0] + wa_ref[p, 1] * a[1] + wa_ref[p, 2] * a[2]
        ea = jnp.where(u, jnp.exp(sa), 0.0)
        ra_ref[p] = (ea / (jnp.sum(ea, axis=-1, keepdims=True) + EPS)).astype(jnp.bfloat16)
        sb = wb_ref[p, 0] * a[0] + wb_ref[p, 1] * a[1] + wb_ref[p, 2] * a[2]
        eb = jnp.where(u, jnp.exp(sb), 0.0)
        rb_ref[p] = (eb / (jnp.sum(eb, axis=-1, keepdims=True) + EPS)).astype(jnp.bfloat16)
        rb2_ref[p] = (wc_ref[p, 0] * a[0] + wc_ref[p, 1] * a[1]
                      + wc_ref[p, 2] * a[2]).astype(jnp.bfloat16)


def _stage1(A, W_a, W_b, W_c, interpret=False):
    big = jax.ShapeDtypeStruct((P, N, N), jnp.bfloat16)
    wspec = pl.BlockSpec(memory_space=pltpu.SMEM)
    return pl.pallas_call(
        _stage1_body,
        grid=(NBLK,),
        in_specs=[
            pl.BlockSpec((R, BM, N), lambda i: (0, i, 0)),
            wspec, wspec, wspec,
        ],
        out_specs=[
            pl.BlockSpec((P, BM, N), lambda i: (0, i, 0)),
            pl.BlockSpec((P, BM, N), lambda i: (0, i, 0)),
            pl.BlockSpec((P, BM, N), lambda i: (0, i, 0)),
        ],
        out_shape=[big, big, big],
        compiler_params=pltpu.CompilerParams(
            dimension_semantics=("parallel",)),
        interpret=interpret,
    )(A, W_a, W_b, W_c)


def _hconv_body(ra_ref, rb_ref, rb2_ref, feat_ref, wg_ref, h_ref):
    i = pl.program_id(1)
    n1 = lax.dot_general(ra_ref[0], rb_ref[0], (((1,), (0,)), ((), ())),
                         preferred_element_type=jnp.float32).astype(jnp.bfloat16)
    n2 = lax.dot_general(n1, rb2_ref[0], (((1,), (0,)), ((), ())),
                         preferred_element_type=jnp.float32).astype(jnp.bfloat16)
    x = jnp.dot(feat_ref[...], wg_ref[0],
                preferred_element_type=jnp.float32,
                precision=lax.Precision.HIGHEST)
    xb = x.astype(jnp.bfloat16)
    col = lax.broadcasted_iota(jnp.int32, (BM, N), 1)
    rowg = lax.broadcasted_iota(jnp.int32, (BM, N), 0) + i * BM
    offdiag = col != rowg

    @pl.when(i == 0)
    def _():
        h_ref[...] = jnp.zeros_like(h_ref)

    for j, m in enumerate((ra_ref[0], n1, n2)):
        c = m.astype(jnp.float32)
        c = jnp.where(offdiag, c, 0.0)
        cn = c / (jnp.sum(c, axis=-1, keepdims=True) + EPS)
        contrib = lax.dot_general(cn.astype(jnp.bfloat16), xb,
                                  (((0,), (0,)), ((), ())),
                                  preferred_element_type=jnp.float32)
        h_ref[0, j] += contrib


def _hconv(RA, RB, RB2, feat, Wg, interpret=False):
    return pl.pallas_call(
        _hconv_body,
        grid=(P, NBLK),
        in_specs=[
            pl.BlockSpec((1, BM, N), lambda p, i: (p, i, 0)),
            pl.BlockSpec((1, N, N), lambda p, i: (p, 0, 0)),
            pl.BlockSpec((1, N, N), lambda p, i: (p, 0, 0)),
            pl.BlockSpec((BM, D_IN), lambda p, i: (i, 0)),
            pl.BlockSpec((1, D_IN, D_H), lambda p, i: (p, 0, 0)),
        ],
        out_specs=pl.BlockSpec((1, 3, N, D_H), lambda p, i: (p, 0, 0, 0)),
        out_shape=jax.ShapeDtypeStruct((P, 3, N, D_H), jnp.float32),
        compiler_params=pltpu.CompilerParams(
            dimension_semantics=("parallel", "arbitrary")),
        interpret=interpret,
    )(RA, RB, RB2, feat, Wg)


def _attn_body(h_ref, bg_ref, law1_ref, lab1_ref, law2_ref,
               maw1_ref, mab1_ref, maw2_ref, out_ref):
    mfs = []
    for p in range(P):
        hs = [jax.nn.relu(h_ref[p, j] + bg_ref[p][None, :]) for j in range(3)]
        cols = []
        for h in hs:
            s = jnp.sum(h * law1_ref[p][None, :], axis=-1, keepdims=True)
            cols.append(jax.nn.relu(jnp.tanh(s + lab1_ref[p]) * law2_ref[p]))
        a = jnp.concatenate(cols, axis=1)
        a = a - jnp.max(a, axis=1, keepdims=True)
        ea = jnp.exp(a)
        b = ea / jnp.sum(ea, axis=1, keepdims=True)
        mf = sum(hs[j] * b[:, j:j + 1] for j in range(3))
        mfs.append(mf)
    mcols = []
    for mf in mfs:
        s = jnp.sum(mf * maw1_ref[0][None, :], axis=-1, keepdims=True)
        mcols.append(jax.nn.relu(jnp.tanh(s + mab1_ref[0]) * maw2_ref[0]))
    ma = jnp.concatenate(mcols, axis=1)
    ma = ma - jnp.max(ma, axis=1, keepdims=True)
    em = jnp.exp(ma)
    mb = em / jnp.sum(em, axis=1, keepdims=True)
    out_ref[...] = sum(mfs[p] * mb[:, p:p + 1] for p in range(P))


BA = 512


def _attn(hacc, bg, la_w1, la_b1, la_w2, ma_w1, ma_b1, ma_w2, interpret=False):
    sspec = pl.BlockSpec(memory_space=pltpu.SMEM)
    return pl.pallas_call(
        _attn_body,
        grid=(N // BA,),
        in_specs=[
            pl.BlockSpec((P, 3, BA, D_H), lambda i: (0, 0, i, 0)),
            pl.BlockSpec((P, D_H), lambda i: (0, 0)),
            pl.BlockSpec((P, D_H), lambda i: (0, 0)),
            sspec,
            sspec,
            pl.BlockSpec((1, D_H), lambda i: (0, 0)),
            sspec,
            sspec,
        ],
        out_specs=pl.BlockSpec((BA, D_H), lambda i: (i, 0)),
        out_shape=jax.ShapeDtypeStruct((N, D_H), jnp.float32),
        compiler_params=pltpu.CompilerParams(
            dimension_semantics=("parallel",)),
        interpret=interpret,
    )(hacc, bg, la_w1, la_b1, la_w2, ma_w1, ma_b1, ma_w2)


WS = 1 << 20
NWIN_A = R * N * N // WS
NTILE = 16
EPT = E // NTILE
NROW = EPT // 128
PT = WS // NTILE
ZCH = 8192
DELTA = 2.0 ** -20


NBUF = 16


def _sc_scatter_body(ei_hbm, ew_hbm, a_hbm,
                     flat_v, w_v, d_v, s_v, iv_bufs, zero_v, win, sem):
    cid = lax.axis_index("c")
    tid = lax.axis_index("s")
    base_e = tid * EPT

    for r in range(R):
        pltpu.sync_copy(ei_hbm.at[pl.ds(r * 2 * E + E + base_e, EPT)], d_v)
        pltpu.sync_copy(ei_hbm.at[pl.ds(r * 2 * E + base_e, EPT)], s_v)
        pltpu.sync_copy(ew_hbm.at[pl.ds(r * E + base_e, EPT)],
                        w_v.at[pl.ds(r * EPT, EPT)])

        @pl.loop(0, EPT // 16)
        def _(c, r=r):
            sl = pl.ds(c * 16, 16)
            fsl = pl.ds(r * EPT + c * 16, 16)
            flat_v[fsl] = d_v[sl] * N + s_v[sl]
            w_v[fsl] = w_v[fsl] + DELTA

    @pl.loop(0, ZCH // 16)
    def _(c):
        zero_v[pl.ds(c * 16, 16)] = jnp.zeros((16,), jnp.float32)

    def zero_window():
        @pl.loop(0, PT // ZCH)
        def _(z):
            pltpu.sync_copy(zero_v, win.at[pl.ds(tid * PT + z * ZCH, ZCH)])

    def stream_rel(rbase, w0):
        @pl.loop(0, NROW, step=NBUF)
        def _(i):
            handles = []
            for b in range(NBUF):
                ib, vb = iv_bufs[2 * b], iv_bufs[2 * b + 1]
                for k in range(8):
                    sl = pl.ds(k * 16, 16)
                    fsl = pl.ds(rbase + (i + b) * 128 + k * 16, 16)
                    adj = flat_v[fsl] - w0
                    ok = (adj >= 0) & (adj < WS)
                    ib[sl] = jnp.where(ok, adj, WS)
                    vb[sl] = w_v[fsl]
                handles.append(pltpu.async_copy(
                    vb, win.at[ib], sem, add=True))
            for h in handles:
                h.wait()

    @pl.loop(0, NWIN_A // 2)
    def _(iw):
        w = cid * (NWIN_A // 2) + iw
        r = w // 4
        q = w - r * 4
        zero_window()
        plsc.subcore_barrier()

        @pl.loop(0, NTILE)
        def _(t):
            @pl.when(tid == t)
            def _():
                stream_rel(r * EPT, q * WS)
            plsc.subcore_barrier()
        pltpu.sync_copy(win.at[pl.ds(tid * PT, PT)],
                        a_hbm.at[pl.ds(w * WS + tid * PT, PT)])


def _scatter_dense(edge_index, edge_weight):
    mesh = plsc.VectorSubcoreMesh(core_axis_name="c", subcore_axis_name="s")
    kern = pl.kernel(
        _sc_scatter_body,
        out_type=jax.ShapeDtypeStruct((R * N * N,), jnp.float32),
        mesh=mesh,
        scratch_types=[
            pltpu.VMEM((R * EPT,), jnp.int32),
            pltpu.VMEM((R * EPT,), jnp.float32),
            pltpu.VMEM((EPT,), jnp.int32),
            pltpu.VMEM((EPT,), jnp.int32),
            [pltpu.VMEM((128,), jnp.int32) if b % 2 == 0
             else pltpu.VMEM((128,), jnp.float32) for b in range(2 * NBUF)],
            pltpu.VMEM((ZCH,), jnp.float32),
            pltpu.VMEM_SHARED((WS + 16,), jnp.float32),
            pltpu.SemaphoreType.DMA,
        ])
    return kern(edge_index.reshape(R * 2 * E), edge_weight.reshape(R * E))


def kernel(feat, edge_index, edge_weight, W_a, W_b, W_c, Wg, bg,
           la_w1, la_b1, la_w2, ma_w1, ma_b1, ma_w2, _interpret=False):
    A_flat = _scatter_dense(edge_index, edge_weight)
    A = A_flat.reshape(R, N, N)
    RA, RB, RB2 = _stage1(A, W_a, W_b, W_c, interpret=_interpret)
    hacc = _hconv(RA, RB, RB2, feat, Wg, interpret=_interpret)
    out = _attn(hacc, bg, la_w1, la_b1, la_w2,
                ma_w1.reshape(1, D_H), ma_b1.reshape(1), ma_w2.reshape(1),
                interpret=_interpret)
    return out
